# Initial kernel scaffold; baseline (speedup 1.0000x reference)
import jax
import jax.numpy as jnp
from jax.experimental import pallas as pl


def kernel(x, scout_W, scout_b, orig_ln_w, orig_ln_b, moe_ln_w, moe_ln_b, W1, b1, W2, b2):
    raise NotImplementedError("write your pallas kernel here")


# speedup vs baseline: 2.8624x; 2.8624x over previous
import functools

import jax
import jax.numpy as jnp
from jax.experimental import pallas as pl
from jax.experimental.pallas import tpu as pltpu

NUM_EXPERTS = 3
TOP_K = 2
D_MODEL = 768
D_FF = 1536
N_TOK = 2048
BLK = 256


def _body(x_ref, swt_ref, sb_ref, olnw_ref, olnb_ref, mlnw_ref, mlnb_ref,
          W1_ref, b1_ref, W2_ref, b2_ref, out_ref, aux_ref, acc_ref):
    i = pl.program_id(0)
    nblk = pl.num_programs(0)
    xb = x_ref[...]

    mu = jnp.mean(xb, axis=1, keepdims=True)
    xc = xb - mu
    var = jnp.mean(xc * xc, axis=1, keepdims=True)
    ln = xc * jax.lax.rsqrt(var + 1e-6)

    logits = []
    for e in range(NUM_EXPERTS):
        w = swt_ref[e:e + 1, :]
        logits.append(jnp.sum(xb * w, axis=1, keepdims=True) + sb_ref[0, e])
    l0, l1, l2 = logits
    m = jnp.maximum(jnp.maximum(l0, l1), l2)
    e0 = jnp.exp(l0 - m)
    e1 = jnp.exp(l1 - m)
    e2 = jnp.exp(l2 - m)
    z = e0 + e1 + e2
    g0, g1, g2 = e0 / z, e1 / z, e2 / z

    x2 = (g2 <= g0) & (g2 <= g1)
    x1 = jnp.logical_not(x2) & (g1 <= g0)
    x0 = jnp.logical_not(x2) & jnp.logical_not(x1)
    gx = jnp.where(x0, g0, jnp.where(x1, g1, g2))
    s = (g0 + g1 + g2) - gx
    inv = 1.0 / (s + 1e-6)
    c0 = jnp.where(x0, 0.0, g0 * inv)
    c1 = jnp.where(x1, 0.0, g1 * inv)
    c2 = jnp.where(x2, 0.0, g2 * inv)

    w_mix = (olnw_ref[...] + g0 * mlnw_ref[0:1, :] + g1 * mlnw_ref[1:2, :]
             + g2 * mlnw_ref[2:3, :])
    b_mix = (olnb_ref[...] + g0 * mlnb_ref[0:1, :] + g1 * mlnb_ref[1:2, :]
             + g2 * mlnb_ref[2:3, :])
    h = ln * w_mix + b_mix

    hb = h.astype(jnp.bfloat16)
    moe = jnp.zeros_like(h)
    combs = (c0, c1, c2)
    for e in range(NUM_EXPERTS):
        t = jnp.dot(hb, W1_ref[e], preferred_element_type=jnp.float32)
        t = t + b1_ref[e:e + 1, :]
        t = jax.nn.gelu(t)
        y = jnp.dot(t.astype(jnp.bfloat16), W2_ref[e],
                    preferred_element_type=jnp.float32)
        y = y + b2_ref[e:e + 1, :]
        moe = moe + combs[e] * y
    out_ref[...] = h + moe

    @pl.when(i == 0)
    def _():
        for k in range(6):
            acc_ref[k] = 0.0

    for e, (g, xe) in enumerate(((g0, x0), (g1, x1), (g2, x2))):
        acc_ref[e] = acc_ref[e] + jnp.sum(g)
        acc_ref[3 + e] = acc_ref[3 + e] + (
            BLK - jnp.sum(xe.astype(jnp.float32)))

    @pl.when(i == nblk - 1)
    def _():
        aux = 0.0
        for e in range(NUM_EXPERTS):
            aux = aux + (acc_ref[3 + e] / N_TOK) * (acc_ref[e] / N_TOK)
        aux_ref[0, 0] = NUM_EXPERTS * aux


@jax.jit
def kernel(x, scout_W, scout_b, orig_ln_w, orig_ln_b, moe_ln_w, moe_ln_b,
           W1, b1, W2, b2):
    n_tok = x.shape[0]
    grid = (n_tok // BLK,)
    out, aux = pl.pallas_call(
        _body,
        grid=grid,
        in_specs=[
            pl.BlockSpec((BLK, D_MODEL), lambda i: (i, 0)),
            pl.BlockSpec((NUM_EXPERTS, D_MODEL), lambda i: (0, 0)),
            pl.BlockSpec((1, NUM_EXPERTS), lambda i: (0, 0)),
            pl.BlockSpec((1, D_MODEL), lambda i: (0, 0)),
            pl.BlockSpec((1, D_MODEL), lambda i: (0, 0)),
            pl.BlockSpec((NUM_EXPERTS, D_MODEL), lambda i: (0, 0)),
            pl.BlockSpec((NUM_EXPERTS, D_MODEL), lambda i: (0, 0)),
            pl.BlockSpec((NUM_EXPERTS, D_MODEL, D_FF), lambda i: (0, 0, 0)),
            pl.BlockSpec((NUM_EXPERTS, D_FF), lambda i: (0, 0)),
            pl.BlockSpec((NUM_EXPERTS, D_FF, D_MODEL), lambda i: (0, 0, 0)),
            pl.BlockSpec((NUM_EXPERTS, D_MODEL), lambda i: (0, 0)),
        ],
        out_specs=[
            pl.BlockSpec((BLK, D_MODEL), lambda i: (i, 0)),
            pl.BlockSpec(memory_space=pltpu.SMEM),
        ],
        out_shape=[
            jax.ShapeDtypeStruct((n_tok, D_MODEL), jnp.float32),
            jax.ShapeDtypeStruct((1, 1), jnp.float32),
        ],
        scratch_shapes=[pltpu.SMEM((8,), jnp.float32)],
        compiler_params=pltpu.CompilerParams(
            dimension_semantics=("arbitrary",)),
    )(
        x, scout_W.T, scout_b.reshape(1, NUM_EXPERTS),
        orig_ln_w.reshape(1, D_MODEL), orig_ln_b.reshape(1, D_MODEL),
        moe_ln_w, moe_ln_b,
        W1.astype(jnp.bfloat16), b1, W2.astype(jnp.bfloat16), b2,
    )
    return out, aux.reshape(())
